# baseline (speedup 1.0000x reference)
import functools

import numpy as np

import jax
import jax.numpy as jnp
from jax import lax
from jax.experimental import pallas as pl
from jax.experimental.pallas import tpu as pltpu
from jax.experimental.pallas import tpu_sc as plsc

_SEQ = 32768
_M = 4096
_MAX_SET = 256
_NUM_BINS = 128
_D_MODEL = 256
_HASH_MULT = 1315423911
_HASH_SEED = 13
_GAMMA = 1.0
_BETA = 0.0

_Z = np.int32(0)
_NC = 2
_NS = 16
_NW = _NC * _NS
_ROWS_PER_TILE = _M // _NW
_IDX_PER_TILE = _ROWS_PER_TILE * _MAX_SET
_PK_PER_TILE = _ROWS_PER_TILE * (_MAX_SET // 2)
_CNT_PER_TILE = _ROWS_PER_TILE * _NUM_BINS


def _hist_sc(tok_i32, idx_flat_i32):
    mesh = plsc.VectorSubcoreMesh(core_axis_name="c", subcore_axis_name="s")

    @functools.partial(
        pl.kernel,
        mesh=mesh,
        out_type=jax.ShapeDtypeStruct((_M * _NUM_BINS,), jnp.float32),
        scratch_types=[
            pltpu.VMEM((_SEQ,), jnp.int32),
            pltpu.VMEM((_PK_PER_TILE,), jnp.int32),
            pltpu.VMEM((_CNT_PER_TILE,), jnp.float32),
            pltpu.SemaphoreType.DMA,
            pltpu.SemaphoreType.DMA,
        ],
        compiler_params=pltpu.CompilerParams(needs_layout_passes=False),
    )
    def hist(tok_hbm, idx_hbm, out_hbm, tok_v, idx_v, cnt_v, sem_t, sem_i):
        i32 = jnp.int32
        wid = lax.axis_index("s") * i32(_NC) + lax.axis_index("c")
        cp_i = pltpu.async_copy(
            idx_hbm.at[pl.ds(wid * i32(_PK_PER_TILE), _PK_PER_TILE)], idx_v, sem_i)
        cp_t = pltpu.async_copy(tok_hbm, tok_v, sem_t)
        cp_t.wait()

        @plsc.parallel_loop(i32(0), i32(_SEQ // 16), i32(1), unroll=8)
        def prehash(k):
            off = k * i32(16)
            t = tok_v[pl.ds(off, 16)]
            tok_v[pl.ds(off, 16)] = (t * i32(_HASH_MULT) + i32(_HASH_SEED)) & i32(_NUM_BINS - 1)

        cp_i.wait()
        ones = jnp.ones((16,), jnp.float32)
        zeros = jnp.zeros((16,), jnp.float32)

        @plsc.parallel_loop(i32(0), i32(_ROWS_PER_TILE), i32(1), unroll=4)
        def row_body(r):
            cbase = r * i32(_NUM_BINS)
            for c in range(_NUM_BINS // 16):
                cnt_v[pl.ds(cbase + i32(c * 16), 16)] = zeros
            ibase = r * i32(_MAX_SET // 2)
            for j in range(_MAX_SET // 32):
                w = idx_v[pl.ds(ibase + i32(j * 16), 16)]
                lo = w & i32(0xFFFF)
                hi = lax.shift_right_logical(w, i32(16))
                bins_lo = plsc.load_gather(tok_v, [lo])
                plsc.addupdate_scatter(cnt_v, [cbase + bins_lo], ones)
                bins_hi = plsc.load_gather(tok_v, [hi])
                plsc.addupdate_scatter(cnt_v, [cbase + bins_hi], ones)
        pltpu.sync_copy(cnt_v, out_hbm.at[pl.ds(wid * i32(_CNT_PER_TILE), _CNT_PER_TILE)])

    return hist(tok_i32, idx_flat_i32)


def _geom_body(prow_ref, pcol_ref, geom_ref):
    geom_ref[...] = -_GAMMA * jnp.abs(prow_ref[...] - pcol_ref[...]) + _BETA


def _geom_tc(prow, pcol):
    blk = 256
    return pl.pallas_call(
        _geom_body,
        grid=(_M // blk,),
        in_specs=[
            pl.BlockSpec((blk, 1), lambda i: (i, _Z)),
            pl.BlockSpec((1, _M), lambda i: (_Z, _Z)),
        ],
        out_specs=pl.BlockSpec((blk, _M), lambda i: (i, _Z)),
        out_shape=jax.ShapeDtypeStruct((_M, _M), jnp.float32),
    )(prow, pcol)


def _feat_body(cnt_ref, sz_ref, wa_ref, ba_ref, wr_ref, br_ref, phi_ref, rout_ref):
    denom = jnp.maximum(sz_ref[...], 1.0)
    c = cnt_ref[...] / denom
    dn = (((1,), (1,)), ((), ()))
    phi_ref[...] = lax.dot_general(c, wa_ref[...], dn,
                                   preferred_element_type=jnp.float32) + ba_ref[...]
    rout_ref[...] = lax.dot_general(c, wr_ref[...], dn,
                                    preferred_element_type=jnp.float32) + br_ref[...]


def _feat_tc(cnt, sizes, w_attn, b_attn, w_router, b_router):
    blk = 2048
    return pl.pallas_call(
        _feat_body,
        grid=(_M // blk,),
        in_specs=[
            pl.BlockSpec((blk, _NUM_BINS), lambda i: (i, _Z)),
            pl.BlockSpec((blk, 1), lambda i: (i, _Z)),
            pl.BlockSpec((_D_MODEL, _NUM_BINS), lambda i: (_Z, _Z)),
            pl.BlockSpec((1, _D_MODEL), lambda i: (_Z, _Z)),
            pl.BlockSpec((_D_MODEL, _NUM_BINS), lambda i: (_Z, _Z)),
            pl.BlockSpec((1, _D_MODEL), lambda i: (_Z, _Z)),
        ],
        out_specs=[
            pl.BlockSpec((blk, _D_MODEL), lambda i: (i, _Z)),
            pl.BlockSpec((blk, _D_MODEL), lambda i: (i, _Z)),
        ],
        out_shape=[
            jax.ShapeDtypeStruct((_M, _D_MODEL), jnp.float32),
            jax.ShapeDtypeStruct((_M, _D_MODEL), jnp.float32),
        ],
    )(cnt, sizes, w_attn, b_attn, w_router, b_router)


def kernel(token_ids, set_indices, set_sizes, set_positions,
           W_attn, b_attn, W_router, b_router):
    tok = token_ids.astype(jnp.int32)
    pairs = set_indices.astype(jnp.int32).reshape(_M, _MAX_SET // 2, 2)
    idx_pk = (pairs[..., 0] | (pairs[..., 1] << 16)).reshape(-1)
    cnt = _hist_sc(tok, idx_pk).reshape(_M, _NUM_BINS)

    sizes = set_sizes.astype(jnp.float32).reshape(_M, 1)
    pos = set_positions.astype(jnp.float32)
    geom = _geom_tc(pos.reshape(_M, 1), pos.reshape(1, _M))
    phi, rout = _feat_tc(cnt, sizes, W_attn.astype(jnp.float32),
                         b_attn.reshape(1, _D_MODEL),
                         W_router.astype(jnp.float32),
                         b_router.reshape(1, _D_MODEL))
    return (phi, rout, geom)


# speedup vs baseline: 1.8771x; 1.8771x over previous
import functools

import numpy as np

import jax
import jax.numpy as jnp
from jax import lax
from jax.experimental import pallas as pl
from jax.experimental.pallas import tpu as pltpu
from jax.experimental.pallas import tpu_sc as plsc

_SEQ = 32768
_M = 4096
_MAX_SET = 256
_NUM_BINS = 128
_D_MODEL = 256
_HASH_MULT = 1315423911
_HASH_SEED = 13
_GAMMA = 1.0
_BETA = 0.0

_Z = np.int32(0)
_NC = 2
_NS = 16
_NW = _NC * _NS
_ROWS_PER_TILE = _M // _NW
_IDX_PER_TILE = _ROWS_PER_TILE * _MAX_SET
_CNT_PER_TILE = _ROWS_PER_TILE * _NUM_BINS


def _hist_sc(tok_i32, idx_flat_i32):
    mesh = plsc.VectorSubcoreMesh(core_axis_name="c", subcore_axis_name="s")

    @functools.partial(
        pl.kernel,
        mesh=mesh,
        out_type=jax.ShapeDtypeStruct((_M * _NUM_BINS,), jnp.float32),
        scratch_types=[
            pltpu.VMEM((_SEQ,), jnp.int32),
            pltpu.VMEM((_IDX_PER_TILE,), jnp.int32),
            pltpu.VMEM((_CNT_PER_TILE,), jnp.float32),
            pltpu.SemaphoreType.DMA,
            pltpu.SemaphoreType.DMA,
        ],
        compiler_params=pltpu.CompilerParams(needs_layout_passes=False),
    )
    def hist(tok_hbm, idx_hbm, out_hbm, tok_v, idx_v, cnt_v, sem_t, sem_i):
        i32 = jnp.int32
        wid = lax.axis_index("s") * i32(_NC) + lax.axis_index("c")
        cp_i = pltpu.async_copy(
            idx_hbm.at[pl.ds(wid * i32(_IDX_PER_TILE), _IDX_PER_TILE)], idx_v, sem_i)
        cp_t = pltpu.async_copy(tok_hbm, tok_v, sem_t)
        cp_t.wait()

        @plsc.parallel_loop(i32(0), i32(_SEQ // 16), i32(1), unroll=8)
        def prehash(k):
            off = k * i32(16)
            t = tok_v[pl.ds(off, 16)]
            tok_v[pl.ds(off, 16)] = (t * i32(_HASH_MULT) + i32(_HASH_SEED)) & i32(_NUM_BINS - 1)

        cp_i.wait()
        ones = jnp.ones((16,), jnp.float32)
        zeros = jnp.zeros((16,), jnp.float32)

        @plsc.parallel_loop(i32(0), i32(_ROWS_PER_TILE), i32(1), unroll=4)
        def row_body(r):
            cbase = r * i32(_NUM_BINS)
            for c in range(_NUM_BINS // 16):
                cnt_v[pl.ds(cbase + i32(c * 16), 16)] = zeros
            ibase = r * i32(_MAX_SET)
            for j in range(_MAX_SET // 16):
                idx = idx_v[pl.ds(ibase + i32(j * 16), 16)]
                bins = plsc.load_gather(tok_v, [idx])
                plsc.addupdate_scatter(cnt_v, [cbase + bins], ones)
        pltpu.sync_copy(cnt_v, out_hbm.at[pl.ds(wid * i32(_CNT_PER_TILE), _CNT_PER_TILE)])

    return hist(tok_i32, idx_flat_i32)


def _geom_body(prow_ref, pcol_ref, geom_ref):
    geom_ref[...] = -_GAMMA * jnp.abs(prow_ref[...] - pcol_ref[...]) + _BETA


def _geom_tc(prow, pcol):
    blk = 256
    return pl.pallas_call(
        _geom_body,
        grid=(_M // blk,),
        in_specs=[
            pl.BlockSpec((blk, 1), lambda i: (i, _Z)),
            pl.BlockSpec((1, _M), lambda i: (_Z, _Z)),
        ],
        out_specs=pl.BlockSpec((blk, _M), lambda i: (i, _Z)),
        out_shape=jax.ShapeDtypeStruct((_M, _M), jnp.float32),
    )(prow, pcol)


def _feat_body(cnt_ref, sz_ref, wa_ref, ba_ref, wr_ref, br_ref, phi_ref, rout_ref):
    denom = jnp.maximum(sz_ref[...], 1.0)
    c = cnt_ref[...] / denom
    dn = (((1,), (1,)), ((), ()))
    phi_ref[...] = lax.dot_general(c, wa_ref[...], dn,
                                   preferred_element_type=jnp.float32) + ba_ref[...]
    rout_ref[...] = lax.dot_general(c, wr_ref[...], dn,
                                    preferred_element_type=jnp.float32) + br_ref[...]


def _feat_tc(cnt, sizes, w_attn, b_attn, w_router, b_router):
    blk = 2048
    return pl.pallas_call(
        _feat_body,
        grid=(_M // blk,),
        in_specs=[
            pl.BlockSpec((blk, _NUM_BINS), lambda i: (i, _Z)),
            pl.BlockSpec((blk, 1), lambda i: (i, _Z)),
            pl.BlockSpec((_D_MODEL, _NUM_BINS), lambda i: (_Z, _Z)),
            pl.BlockSpec((1, _D_MODEL), lambda i: (_Z, _Z)),
            pl.BlockSpec((_D_MODEL, _NUM_BINS), lambda i: (_Z, _Z)),
            pl.BlockSpec((1, _D_MODEL), lambda i: (_Z, _Z)),
        ],
        out_specs=[
            pl.BlockSpec((blk, _D_MODEL), lambda i: (i, _Z)),
            pl.BlockSpec((blk, _D_MODEL), lambda i: (i, _Z)),
        ],
        out_shape=[
            jax.ShapeDtypeStruct((_M, _D_MODEL), jnp.float32),
            jax.ShapeDtypeStruct((_M, _D_MODEL), jnp.float32),
        ],
    )(cnt, sizes, w_attn, b_attn, w_router, b_router)


def kernel(token_ids, set_indices, set_sizes, set_positions,
           W_attn, b_attn, W_router, b_router):
    pos = set_positions.astype(jnp.float32)
    geom = _geom_tc(pos.reshape(_M, 1), pos.reshape(1, _M))

    tok = token_ids.astype(jnp.int32)
    idx_flat = set_indices.astype(jnp.int32).reshape(-1)
    cnt = _hist_sc(tok, idx_flat).reshape(_M, _NUM_BINS)

    sizes = set_sizes.astype(jnp.float32).reshape(_M, 1)
    phi, rout = _feat_tc(cnt, sizes, W_attn.astype(jnp.float32),
                         b_attn.reshape(1, _D_MODEL),
                         W_router.astype(jnp.float32),
                         b_router.reshape(1, _D_MODEL))
    return (phi, rout, geom)
